# Initial kernel scaffold; baseline (speedup 1.0000x reference)
import jax
import jax.numpy as jnp
from jax.experimental import pallas as pl


def kernel(y_true, y_pred):
    raise NotImplementedError("write your pallas kernel here")


# speedup vs baseline: 313.0759x; 313.0759x over previous
import jax
import jax.numpy as jnp
from jax.experimental import pallas as pl
from jax.experimental.pallas import tpu as pltpu

B, C, H, W = 16, 4, 512, 512
N_HW = H * W
ROWS = 8
NCHUNK = H // ROWS
NEG_POS = 3
ONE_BITS = 0x3F800000


def _sl1(d):
    ad = jnp.abs(d)
    return jnp.where(ad < 1.0, 0.5 * d * d, ad - 0.5)


def _cumsum(x, axis):
    n = x.shape[axis]
    s = 1
    while s < n:
        pad = jnp.zeros_like(jax.lax.slice_in_dim(x, 0, s, axis=axis))
        shifted = jnp.concatenate(
            [pad, jax.lax.slice_in_dim(x, 0, n - s, axis=axis)], axis=axis)
        x = x + shifted
        s *= 2
    return x


def _stats_body(yt_ref, yp_ref, p_ref, z_ref, ps_ref, nz_ref):
    yt = yt_ref[...]
    yp = yp_ref[...]
    d = yt - yp
    sl1 = _sl1(d)
    pos = yt >= 0.5
    zero = jnp.logical_or(pos, d == 0.0)
    p_ref[0] = jnp.sum(pos.astype(jnp.float32), axis=(2, 3))
    z_ref[0] = jnp.sum(zero.astype(jnp.float32), axis=(2, 3))
    ps_ref[0] = jnp.sum(jnp.where(pos, sl1, 0.0), axis=(2, 3))
    nz_ref[0] = jnp.sum(jnp.where(zero, 0.0, sl1), axis=(2, 3))


def _partial_body(bidx_ref, r_ref, yt_ref, yp_ref, out_ref):
    i = pl.program_id(0)

    @pl.when(i == 0)
    def _():
        out_ref[...] = jnp.zeros_like(out_ref)

    yt = yt_ref[0, 0]
    yp = yp_ref[0, 0]
    d = yt - yp
    pos = yt >= 0.5
    zero = jnp.logical_or(pos, d == 0.0)
    zf = zero.astype(jnp.float32)
    c = _cumsum(zf, axis=1)
    rowtot = c[:, W - 1:W]
    rowcum = _cumsum(rowtot, axis=0) - rowtot
    crank = c + rowcum
    r = r_ref[i].astype(jnp.float32)
    sel = jnp.logical_and(zero, crank <= r)
    val = jnp.sum(jnp.where(sel, _sl1(d), 0.0))
    row_i = jax.lax.broadcasted_iota(jnp.int32, (B, C), 0)
    col_i = jax.lax.broadcasted_iota(jnp.int32, (B, C), 1)
    mask = jnp.logical_and(row_i == i // C, col_i == i % C)
    out_ref[...] = out_ref[...] + jnp.where(mask, val, 0.0)


def _count_body(t_ref, yt_ref, yp_ref, cnt_ref):
    j = pl.program_id(0)

    @pl.when(j == 0)
    def _():
        cnt_ref[...] = jnp.zeros_like(cnt_ref)

    yt = yt_ref[...]
    yp = yp_ref[...]
    pos = yt >= 0.5
    loss = jnp.where(pos, 0.0, jnp.abs(yt - yp))
    t = t_ref[...][:, :, None, None]
    cnt_ref[...] += jnp.sum((loss > t).astype(jnp.float32), axis=(2, 3))


def _gt_body(t_ref, yt_ref, yp_ref, cnt_ref, sum_ref):
    j = pl.program_id(0)

    @pl.when(j == 0)
    def _():
        cnt_ref[...] = jnp.zeros_like(cnt_ref)
        sum_ref[...] = jnp.zeros_like(sum_ref)

    yt = yt_ref[...]
    yp = yp_ref[...]
    d = yt - yp
    pos = yt >= 0.5
    loss = jnp.where(pos, 0.0, jnp.abs(d))
    t = t_ref[...][:, :, None, None]
    gt = loss > t
    cnt_ref[...] += jnp.sum(gt.astype(jnp.float32), axis=(2, 3))
    sum_ref[...] += jnp.sum(jnp.where(gt, _sl1(d), 0.0), axis=(2, 3))


_BLOCK4D = pl.BlockSpec((B, C, ROWS, W), lambda j: (0, 0, j, 0))
_STATS_OUT = pl.BlockSpec((1, B, C), lambda j: (j, 0, 0))
_BC_IN = pl.BlockSpec((B, C), lambda j: (0, 0))
_BC_OUT = pl.BlockSpec((B, C), lambda j: (0, 0))


def _count_gt(yt, yp, t):
    return pl.pallas_call(
        _count_body,
        grid=(NCHUNK,),
        in_specs=[_BC_IN, _BLOCK4D, _BLOCK4D],
        out_specs=_BC_OUT,
        out_shape=jax.ShapeDtypeStruct((B, C), jnp.float32),
    )(t, yt, yp)


def _atypical_sums(yt, yp, k):
    kf = k.astype(jnp.float32)

    def body(_, carry):
        lo, hi = carry
        mid = jnp.where(lo < hi, (lo + hi) // 2, lo)
        t = jax.lax.bitcast_convert_type(mid, jnp.float32)
        cnt = _count_gt(yt, yp, t)
        less = cnt < kf
        lo2 = jnp.where(jnp.logical_and(lo < hi, jnp.logical_not(less)),
                        mid + 1, lo)
        hi2 = jnp.where(jnp.logical_and(lo < hi, less), mid, hi)
        return lo2, hi2

    lo = jnp.zeros_like(k)
    hi = jnp.full_like(k, ONE_BITS)
    lo, _ = jax.lax.fori_loop(0, 31, body, (lo, hi))
    t = jax.lax.bitcast_convert_type(lo, jnp.float32)
    cnt, gsum = pl.pallas_call(
        _gt_body,
        grid=(NCHUNK,),
        in_specs=[_BC_IN, _BLOCK4D, _BLOCK4D],
        out_specs=[_BC_OUT, _BC_OUT],
        out_shape=[jax.ShapeDtypeStruct((B, C), jnp.float32),
                   jax.ShapeDtypeStruct((B, C), jnp.float32)],
    )(t, yt, yp)
    sl1_t = jnp.where(t < 1.0, 0.5 * t * t, t - 0.5)
    return gsum + (kf - cnt) * sl1_t


def kernel(y_true, y_pred):
    yt = y_true
    yp = y_pred

    p_c, z_c, ps_c, nz_c = pl.pallas_call(
        _stats_body,
        grid=(NCHUNK,),
        in_specs=[_BLOCK4D, _BLOCK4D],
        out_specs=[_STATS_OUT] * 4,
        out_shape=[jax.ShapeDtypeStruct((NCHUNK, B, C), jnp.float32)] * 4,
    )(yt, yp)

    p_tot = jnp.sum(p_c, axis=0)
    z_tot = jnp.sum(z_c, axis=0)
    ps_tot = jnp.sum(ps_c, axis=0)
    nz_tot = jnp.sum(nz_c, axis=0)

    p_i = p_tot.astype(jnp.int32)
    z_i = z_tot.astype(jnp.int32)
    k = jnp.minimum(NEG_POS * p_i, N_HW - 1)
    g = N_HW - z_i
    q = k - g
    typical = q >= 0

    cz = jnp.cumsum(z_c, axis=0)
    qf = q.astype(jnp.float32)
    sum_full = jnp.sum(jnp.where(cz <= qf[None], ps_c, 0.0), axis=0)
    bidx = jnp.argmax(cz > qf[None], axis=0).astype(jnp.int32)
    cz_ex_b = jnp.take_along_axis(cz - z_c, bidx[None], axis=0)[0]
    r = jnp.where(typical, q - cz_ex_b.astype(jnp.int32), 0)
    r = jnp.maximum(r, 0)

    partial = pl.pallas_call(
        _partial_body,
        grid_spec=pltpu.PrefetchScalarGridSpec(
            num_scalar_prefetch=2,
            grid=(B * C,),
            in_specs=[
                pl.BlockSpec((1, 1, ROWS, W),
                             lambda i, b_, r_: (i // C, i % C, b_[i], 0)),
                pl.BlockSpec((1, 1, ROWS, W),
                             lambda i, b_, r_: (i // C, i % C, b_[i], 0)),
            ],
            out_specs=pl.BlockSpec((B, C), lambda i, b_, r_: (0, 0)),
        ),
        out_shape=jax.ShapeDtypeStruct((B, C), jnp.float32),
    )(bidx.reshape(B * C), r.reshape(B * C), yt, yp)

    atyp = jax.lax.cond(
        jnp.any(jnp.logical_not(typical)),
        lambda: _atypical_sums(yt, yp, k),
        lambda: jnp.zeros((B, C), jnp.float32),
    )

    neg_sum = jnp.where(typical, nz_tot + sum_full + partial, atyp)
    pos_cnt = jnp.maximum(jnp.sum(p_tot), 1.0)
    neg_cnt = jnp.maximum(jnp.sum(k.astype(jnp.float32)), 1.0)
    pos_loss = jnp.sum(ps_tot) / pos_cnt
    neg_loss = jnp.sum(neg_sum) / neg_cnt
    return NEG_POS * pos_loss + neg_loss


# speedup vs baseline: 472.1509x; 1.5081x over previous
import jax
import jax.numpy as jnp
from jax.experimental import pallas as pl
from jax.experimental.pallas import tpu as pltpu

B, C, H, W = 16, 4, 512, 512
N_HW = H * W
ROWS = 16
NCHUNK = H // ROWS
NEG_POS = 3
ONE_BITS = 0x3F800000


def _fold_lanes(x):
    return x[..., 0:128] + x[..., 128:256] + x[..., 256:384] + x[..., 384:512]


def _cumsum(x, axis):
    n = x.shape[axis]
    s = 1
    while s < n:
        pad = jnp.zeros_like(jax.lax.slice_in_dim(x, 0, s, axis=axis))
        shifted = jnp.concatenate(
            [pad, jax.lax.slice_in_dim(x, 0, n - s, axis=axis)], axis=axis)
        x = x + shifted
        s *= 2
    return x


def _stats_body(yt_ref, yp_ref, z_ref, ps_ref, p_ref, all_ref,
                acc_p, acc_all):
    j = pl.program_id(0)

    @pl.when(j == 0)
    def _():
        acc_p[...] = jnp.zeros_like(acc_p)
        acc_all[...] = jnp.zeros_like(acc_all)

    yt = yt_ref[...]
    yp = yp_ref[...]
    d = yt - yp
    d2 = d * d
    pos = yt >= 0.5
    zero = jnp.logical_or(pos, d2 == 0.0)
    z_ref[0] = jnp.sum(zero.astype(jnp.float32), axis=(2, 3))
    ps_ref[0] = jnp.sum(jnp.where(pos, d2, 0.0), axis=(2, 3))
    pf = _fold_lanes(jnp.where(pos, 1.0, 0.0))
    af = _fold_lanes(d2)
    acc_p[...] += pf[:, :, 0:ROWS // 2] + pf[:, :, ROWS // 2:ROWS]
    acc_all[...] += af[:, :, 0:ROWS // 2] + af[:, :, ROWS // 2:ROWS]

    @pl.when(j == NCHUNK - 1)
    def _():
        p_ref[...] = jnp.sum(acc_p[...], axis=(2, 3))
        all_ref[...] = jnp.sum(acc_all[...], axis=(2, 3))


def _plan_body(z_ref, ps_ref, p_ref, all_ref,
               bidx_ref, r_ref, kf_ref, typb_ref, typm_ref, misc_ref):
    zc = z_ref[...]
    psc = ps_ref[...]
    p = p_ref[...]
    alls = all_ref[...]

    ps_tot = jnp.sum(psc, axis=0)
    z_tot = jnp.sum(zc, axis=0)
    k = jnp.minimum(3.0 * p, float(N_HW - 1))
    g = float(N_HW) - z_tot
    q = k - g
    typ = q >= 0.0
    nz_tot = alls - ps_tot

    cz = _cumsum(zc, axis=0)
    qb = q[None]
    le = cz <= qb
    sum_full = jnp.sum(jnp.where(le, psc, 0.0), axis=0)
    bidx = jnp.sum(jnp.where(le, 1.0, 0.0), axis=0)
    cz_ex_b = jnp.max(jnp.where(le, cz, 0.0), axis=0)
    r = jnp.where(typ, q - cz_ex_b, 0.0)

    bidx_ref[...] = bidx.astype(jnp.int32)
    r_ref[...] = r.astype(jnp.int32)
    kf_ref[...] = k
    typb_ref[...] = jnp.where(typ, 0.5 * (nz_tot + sum_full), 0.0)
    typm_ref[...] = jnp.where(typ, 1.0, 0.0)

    row = jax.lax.broadcasted_iota(jnp.int32, (B, C), 0)
    col = jax.lax.broadcasted_iota(jnp.int32, (B, C), 1)
    ptot = jnp.sum(p)
    pstot = 0.5 * jnp.sum(ps_tot)
    ktot = jnp.sum(k)
    natyp = jnp.sum(jnp.where(typ, 0.0, 1.0))
    misc = jnp.where(jnp.logical_and(row == 0, col == 0), ptot, 0.0)
    misc = jnp.where(jnp.logical_and(row == 0, col == 1), pstot, misc)
    misc = jnp.where(jnp.logical_and(row == 0, col == 2), ktot, misc)
    misc = jnp.where(jnp.logical_and(row == 0, col == 3), natyp, misc)
    misc_ref[...] = misc


def _partial_body(bidx_ref, r_ref, *refs):
    out_ref = refs[-1]
    acc = jnp.zeros((B, C), jnp.float32)
    row = jax.lax.broadcasted_iota(jnp.int32, (B, C), 0)
    col = jax.lax.broadcasted_iota(jnp.int32, (B, C), 1)
    for m in range(B * C):
        yt = refs[m][0, 0]
        yp = refs[B * C + m][0, 0]
        d = yt - yp
        d2 = d * d
        pos = yt >= 0.5
        zero = jnp.logical_or(pos, d2 == 0.0)
        zf = zero.astype(jnp.float32)
        c = _cumsum(zf, axis=1)
        rowtot = c[:, W - 1:W]
        rowcum = _cumsum(rowtot, axis=0) - rowtot
        crank = c + rowcum
        r = r_ref[m // C, m % C].astype(jnp.float32)
        sel = jnp.logical_and(zero, crank <= r)
        val = 0.5 * jnp.sum(jnp.where(sel, d2, 0.0))
        mask = jnp.logical_and(row == m // C, col == m % C)
        acc = acc + jnp.where(mask, val, 0.0)
    out_ref[...] = acc


def _count_body(t_ref, yt_ref, yp_ref, cnt_ref):
    j = pl.program_id(0)

    @pl.when(j == 0)
    def _():
        cnt_ref[...] = jnp.zeros_like(cnt_ref)

    yt = yt_ref[...]
    yp = yp_ref[...]
    pos = yt >= 0.5
    loss = jnp.where(pos, 0.0, jnp.abs(yt - yp))
    t = t_ref[...][:, :, None, None]
    cnt_ref[...] += jnp.sum((loss > t).astype(jnp.float32), axis=(2, 3))


def _gt_body(t_ref, yt_ref, yp_ref, cnt_ref, sum_ref):
    j = pl.program_id(0)

    @pl.when(j == 0)
    def _():
        cnt_ref[...] = jnp.zeros_like(cnt_ref)
        sum_ref[...] = jnp.zeros_like(sum_ref)

    yt = yt_ref[...]
    yp = yp_ref[...]
    d = yt - yp
    pos = yt >= 0.5
    loss = jnp.where(pos, 0.0, jnp.abs(d))
    t = t_ref[...][:, :, None, None]
    gt = loss > t
    cnt_ref[...] += jnp.sum(gt.astype(jnp.float32), axis=(2, 3))
    sum_ref[...] += jnp.sum(jnp.where(gt, 0.5 * d * d, 0.0), axis=(2, 3))


_BLOCK4D = pl.BlockSpec((B, C, ROWS, W), lambda j: (0, 0, j, 0))
_BC_IN = pl.BlockSpec((B, C), lambda j: (0, 0))
_BC_OUT = pl.BlockSpec((B, C), lambda j: (0, 0))


def _count_gt(yt, yp, t):
    return pl.pallas_call(
        _count_body,
        grid=(NCHUNK,),
        in_specs=[_BC_IN, _BLOCK4D, _BLOCK4D],
        out_specs=_BC_OUT,
        out_shape=jax.ShapeDtypeStruct((B, C), jnp.float32),
    )(t, yt, yp)


def _atypical_sums(yt, yp, kf):
    k = kf.astype(jnp.int32)

    def body(_, carry):
        lo, hi = carry
        mid = jnp.where(lo < hi, (lo + hi) // 2, lo)
        t = jax.lax.bitcast_convert_type(mid, jnp.float32)
        cnt = _count_gt(yt, yp, t)
        less = cnt < kf
        lo2 = jnp.where(jnp.logical_and(lo < hi, jnp.logical_not(less)),
                        mid + 1, lo)
        hi2 = jnp.where(jnp.logical_and(lo < hi, less), mid, hi)
        return lo2, hi2

    lo = jnp.zeros_like(k)
    hi = jnp.full_like(k, ONE_BITS)
    lo, _ = jax.lax.fori_loop(0, 31, body, (lo, hi))
    t = jax.lax.bitcast_convert_type(lo, jnp.float32)
    cnt, gsum = pl.pallas_call(
        _gt_body,
        grid=(NCHUNK,),
        in_specs=[_BC_IN, _BLOCK4D, _BLOCK4D],
        out_specs=[_BC_OUT, _BC_OUT],
        out_shape=[jax.ShapeDtypeStruct((B, C), jnp.float32),
                   jax.ShapeDtypeStruct((B, C), jnp.float32)],
    )(t, yt, yp)
    sl1_t = jnp.where(t < 1.0, 0.5 * t * t, t - 0.5)
    return gsum + (kf - cnt) * sl1_t


def _partial_spec(m):
    return pl.BlockSpec(
        (1, 1, ROWS, W),
        lambda i, b_, r_, m=m: (m // C, m % C, b_[m // C, m % C], 0))


def kernel(y_true, y_pred):
    yt = y_true
    yp = y_pred

    z_c, ps_c, p_s, all_s = pl.pallas_call(
        _stats_body,
        grid=(NCHUNK,),
        in_specs=[_BLOCK4D, _BLOCK4D],
        out_specs=[pl.BlockSpec((1, B, C), lambda j: (j, 0, 0)),
                   pl.BlockSpec((1, B, C), lambda j: (j, 0, 0)),
                   _BC_OUT, _BC_OUT],
        out_shape=[jax.ShapeDtypeStruct((NCHUNK, B, C), jnp.float32),
                   jax.ShapeDtypeStruct((NCHUNK, B, C), jnp.float32),
                   jax.ShapeDtypeStruct((B, C), jnp.float32),
                   jax.ShapeDtypeStruct((B, C), jnp.float32)],
        scratch_shapes=[
            pltpu.VMEM((B, C, ROWS // 2, 128), jnp.float32),
            pltpu.VMEM((B, C, ROWS // 2, 128), jnp.float32),
        ],
    )(yt, yp)

    full_spec = pl.BlockSpec((NCHUNK, B, C), lambda: (0, 0, 0))
    bidx, r, kf, typb, typm, misc = pl.pallas_call(
        _plan_body,
        in_specs=[full_spec, full_spec,
                  pl.BlockSpec((B, C), lambda: (0, 0)),
                  pl.BlockSpec((B, C), lambda: (0, 0))],
        out_specs=[pl.BlockSpec((B, C), lambda: (0, 0))] * 6,
        out_shape=[jax.ShapeDtypeStruct((B, C), jnp.int32),
                   jax.ShapeDtypeStruct((B, C), jnp.int32)] +
                  [jax.ShapeDtypeStruct((B, C), jnp.float32)] * 4,
    )(z_c, ps_c, p_s, all_s)

    partial = pl.pallas_call(
        _partial_body,
        grid_spec=pltpu.PrefetchScalarGridSpec(
            num_scalar_prefetch=2,
            grid=(1,),
            in_specs=[_partial_spec(m) for m in range(B * C)] * 2,
            out_specs=pl.BlockSpec((B, C), lambda i, b_, r_: (0, 0)),
        ),
        out_shape=jax.ShapeDtypeStruct((B, C), jnp.float32),
    )(bidx, r, *([yt] * (B * C)), *([yp] * (B * C)))

    atyp = jax.lax.cond(
        misc[0, 3] > 0.0,
        lambda: _atypical_sums(yt, yp, kf),
        lambda: jnp.zeros((B, C), jnp.float32),
    )

    neg_sum = jnp.where(typm > 0.0, typb + partial, atyp)
    pos_cnt = jnp.maximum(misc[0, 0], 1.0)
    neg_cnt = jnp.maximum(misc[0, 2], 1.0)
    neg_loss = jnp.sum(neg_sum) / neg_cnt
    return NEG_POS * (misc[0, 1] / pos_cnt) + neg_loss


# speedup vs baseline: 833.9525x; 1.7663x over previous
import jax
import jax.numpy as jnp
from jax.experimental import pallas as pl
from jax.experimental.pallas import tpu as pltpu

B, C, H, W = 16, 4, 512, 512
N_HW = H * W
ROWS = 16
NCHUNK = H // ROWS
NEG_POS = 3
ONE_BITS = 0x3F800000


def _fold_lanes(x):
    return x[..., 0:128] + x[..., 128:256] + x[..., 256:384] + x[..., 384:512]


def _cumsum(x, axis):
    n = x.shape[axis]
    s = 1
    while s < n:
        pad = jnp.zeros_like(jax.lax.slice_in_dim(x, 0, s, axis=axis))
        shifted = jnp.concatenate(
            [pad, jax.lax.slice_in_dim(x, 0, n - s, axis=axis)], axis=axis)
        x = x + shifted
        s *= 2
    return x


def _stats_body(yt_ref, yp_ref,
                bidx_ref, rf_ref, kf_ref, typb_ref, typm_ref, misc_ref,
                z_s, ps_s, acc_p, acc_all):
    j = pl.program_id(0)

    @pl.when(j == 0)
    def _():
        acc_p[...] = jnp.zeros_like(acc_p)
        acc_all[...] = jnp.zeros_like(acc_all)

    yt = yt_ref[...]
    yp = yp_ref[...]
    d = yt - yp
    d2 = d * d
    pos = yt >= 0.5
    posf = jnp.where(pos, 1.0, 0.0)
    zerof = jnp.where(d2 == 0.0, 1.0, posf)
    z_s[j] = jnp.sum(zerof, axis=(2, 3))
    ps_s[j] = jnp.sum(jnp.where(pos, d2, 0.0), axis=(2, 3))
    pf = _fold_lanes(posf)
    af = _fold_lanes(d2)
    acc_p[...] += pf[:, :, 0:ROWS // 2] + pf[:, :, ROWS // 2:ROWS]
    acc_all[...] += af[:, :, 0:ROWS // 2] + af[:, :, ROWS // 2:ROWS]

    @pl.when(j == NCHUNK - 1)
    def _():
        p = jnp.sum(acc_p[...], axis=(2, 3))
        alls = jnp.sum(acc_all[...], axis=(2, 3))
        zc = z_s[...]
        psc = ps_s[...]
        ps_tot = jnp.sum(psc, axis=0)
        z_tot = jnp.sum(zc, axis=0)
        k = jnp.minimum(3.0 * p, float(N_HW - 1))
        g = float(N_HW) - z_tot
        q = k - g
        typ = q >= 0.0
        nz_tot = alls - ps_tot
        cz = _cumsum(zc, axis=0)
        le = cz <= q[None]
        sum_full = jnp.sum(jnp.where(le, psc, 0.0), axis=0)
        bidx = jnp.sum(jnp.where(le, 1.0, 0.0), axis=0)
        cz_ex_b = jnp.max(jnp.where(le, cz, 0.0), axis=0)
        r = jnp.where(typ, q - cz_ex_b, 0.0)

        bidx_ref[...] = bidx.astype(jnp.int32)
        rf_ref[...] = r
        kf_ref[...] = k
        typb_ref[...] = jnp.where(typ, 0.5 * (nz_tot + sum_full), 0.0)
        typm_ref[...] = jnp.where(typ, 1.0, 0.0)

        row = jax.lax.broadcasted_iota(jnp.int32, (B, C), 0)
        col = jax.lax.broadcasted_iota(jnp.int32, (B, C), 1)
        misc = jnp.where(jnp.logical_and(row == 0, col == 0), jnp.sum(p), 0.0)
        misc = jnp.where(jnp.logical_and(row == 0, col == 1),
                         0.5 * jnp.sum(ps_tot), misc)
        misc = jnp.where(jnp.logical_and(row == 0, col == 2),
                         jnp.sum(k), misc)
        misc = jnp.where(jnp.logical_and(row == 0, col == 3),
                         jnp.sum(jnp.where(typ, 0.0, 1.0)), misc)
        misc_ref[...] = misc


def _partial_body(bidx_ref, rf_ref, typb_ref, misc_ref, yt_hbm, yp_hbm,
                  pvec_ref, scal_ref, syt, syp, sem_t, sem_p):
    for m in range(B * C):
        b = m // C
        c = m % C
        off = bidx_ref[b, c] * ROWS
        pltpu.make_async_copy(
            yt_hbm.at[b, c, pl.ds(off, ROWS), :], syt.at[b, c],
            sem_t.at[b, c]).start()
        pltpu.make_async_copy(
            yp_hbm.at[b, c, pl.ds(off, ROWS), :], syp.at[b, c],
            sem_p.at[b, c]).start()
    for m in range(B * C):
        b = m // C
        c = m % C
        off = bidx_ref[b, c] * ROWS
        pltpu.make_async_copy(
            yt_hbm.at[b, c, pl.ds(off, ROWS), :], syt.at[b, c],
            sem_t.at[b, c]).wait()
        pltpu.make_async_copy(
            yp_hbm.at[b, c, pl.ds(off, ROWS), :], syp.at[b, c],
            sem_p.at[b, c]).wait()

    yt = syt[...]
    yp = syp[...]
    d = yt - yp
    d2 = d * d
    pos = yt >= 0.5
    zero = jnp.logical_or(pos, d2 == 0.0)
    zf = jnp.where(zero, 1.0, 0.0)
    cs = _cumsum(zf, axis=3)
    rowtot = cs[:, :, :, W - 1:W]
    rowcum = _cumsum(rowtot, axis=2) - rowtot
    crank = cs + rowcum
    rv = rf_ref[...][:, :, None, None]
    sel = jnp.logical_and(zero, crank <= rv)
    pvec = 0.5 * jnp.sum(jnp.where(sel, d2, 0.0), axis=(2, 3))
    pvec_ref[...] = pvec

    misc = misc_ref[...]
    neg_sum = jnp.sum(typb_ref[...] + pvec)
    pos_cnt = jnp.maximum(misc[0, 0], 1.0)
    neg_cnt = jnp.maximum(misc[0, 2], 1.0)
    out = NEG_POS * (misc[0, 1] / pos_cnt) + neg_sum / neg_cnt
    scal_ref[...] = jnp.full((1, 1), 1.0) * out


def _count_body(t_ref, yt_ref, yp_ref, cnt_ref):
    j = pl.program_id(0)

    @pl.when(j == 0)
    def _():
        cnt_ref[...] = jnp.zeros_like(cnt_ref)

    yt = yt_ref[...]
    yp = yp_ref[...]
    pos = yt >= 0.5
    loss = jnp.where(pos, 0.0, jnp.abs(yt - yp))
    t = t_ref[...][:, :, None, None]
    cnt_ref[...] += jnp.sum((loss > t).astype(jnp.float32), axis=(2, 3))


def _gt_body(t_ref, yt_ref, yp_ref, cnt_ref, sum_ref):
    j = pl.program_id(0)

    @pl.when(j == 0)
    def _():
        cnt_ref[...] = jnp.zeros_like(cnt_ref)
        sum_ref[...] = jnp.zeros_like(sum_ref)

    yt = yt_ref[...]
    yp = yp_ref[...]
    d = yt - yp
    pos = yt >= 0.5
    loss = jnp.where(pos, 0.0, jnp.abs(d))
    t = t_ref[...][:, :, None, None]
    gt = loss > t
    cnt_ref[...] += jnp.sum(gt.astype(jnp.float32), axis=(2, 3))
    sum_ref[...] += jnp.sum(jnp.where(gt, 0.5 * d * d, 0.0), axis=(2, 3))


_BLOCK4D = pl.BlockSpec((B, C, ROWS, W), lambda j: (0, 0, j, 0))
_BC_IN = pl.BlockSpec((B, C), lambda j: (0, 0))
_BC_OUT = pl.BlockSpec((B, C), lambda j: (0, 0))


def _count_gt(yt, yp, t):
    return pl.pallas_call(
        _count_body,
        grid=(NCHUNK,),
        in_specs=[_BC_IN, _BLOCK4D, _BLOCK4D],
        out_specs=_BC_OUT,
        out_shape=jax.ShapeDtypeStruct((B, C), jnp.float32),
    )(t, yt, yp)


def _atypical_sums(yt, yp, kf):
    k = kf.astype(jnp.int32)

    def body(_, carry):
        lo, hi = carry
        mid = jnp.where(lo < hi, (lo + hi) // 2, lo)
        t = jax.lax.bitcast_convert_type(mid, jnp.float32)
        cnt = _count_gt(yt, yp, t)
        less = cnt < kf
        lo2 = jnp.where(jnp.logical_and(lo < hi, jnp.logical_not(less)),
                        mid + 1, lo)
        hi2 = jnp.where(jnp.logical_and(lo < hi, less), mid, hi)
        return lo2, hi2

    lo = jnp.zeros_like(k)
    hi = jnp.full_like(k, ONE_BITS)
    lo, _ = jax.lax.fori_loop(0, 31, body, (lo, hi))
    t = jax.lax.bitcast_convert_type(lo, jnp.float32)
    cnt, gsum = pl.pallas_call(
        _gt_body,
        grid=(NCHUNK,),
        in_specs=[_BC_IN, _BLOCK4D, _BLOCK4D],
        out_specs=[_BC_OUT, _BC_OUT],
        out_shape=[jax.ShapeDtypeStruct((B, C), jnp.float32),
                   jax.ShapeDtypeStruct((B, C), jnp.float32)],
    )(t, yt, yp)
    sl1_t = jnp.where(t < 1.0, 0.5 * t * t, t - 0.5)
    return gsum + (kf - cnt) * sl1_t


def kernel(y_true, y_pred):
    yt = y_true
    yp = y_pred

    bidx, rf, kf, typb, typm, misc = pl.pallas_call(
        _stats_body,
        grid=(NCHUNK,),
        in_specs=[_BLOCK4D, _BLOCK4D],
        out_specs=[_BC_OUT] * 6,
        out_shape=[jax.ShapeDtypeStruct((B, C), jnp.int32)] +
                  [jax.ShapeDtypeStruct((B, C), jnp.float32)] * 5,
        scratch_shapes=[
            pltpu.VMEM((NCHUNK, B, C), jnp.float32),
            pltpu.VMEM((NCHUNK, B, C), jnp.float32),
            pltpu.VMEM((B, C, ROWS // 2, 128), jnp.float32),
            pltpu.VMEM((B, C, ROWS // 2, 128), jnp.float32),
        ],
    )(yt, yp)

    pvec, scal = pl.pallas_call(
        _partial_body,
        grid_spec=pltpu.PrefetchScalarGridSpec(
            num_scalar_prefetch=1,
            grid=(1,),
            in_specs=[
                pl.BlockSpec((B, C), lambda i, b_: (0, 0)),
                pl.BlockSpec((B, C), lambda i, b_: (0, 0)),
                pl.BlockSpec((B, C), lambda i, b_: (0, 0)),
                pl.BlockSpec(memory_space=pl.ANY),
                pl.BlockSpec(memory_space=pl.ANY),
            ],
            out_specs=[pl.BlockSpec((B, C), lambda i, b_: (0, 0)),
                       pl.BlockSpec((1, 1), lambda i, b_: (0, 0))],
            scratch_shapes=[
                pltpu.VMEM((B, C, ROWS, W), jnp.float32),
                pltpu.VMEM((B, C, ROWS, W), jnp.float32),
                pltpu.SemaphoreType.DMA((B, C)),
                pltpu.SemaphoreType.DMA((B, C)),
            ],
        ),
        out_shape=[jax.ShapeDtypeStruct((B, C), jnp.float32),
                   jax.ShapeDtypeStruct((1, 1), jnp.float32)],
    )(bidx, rf, typb, misc, yt, yp)

    def _atyp_path():
        atyp = _atypical_sums(yt, yp, kf)
        neg_sum = jnp.sum(jnp.where(typm > 0.0, typb + pvec, atyp))
        pos_cnt = jnp.maximum(misc[0, 0], 1.0)
        neg_cnt = jnp.maximum(misc[0, 2], 1.0)
        return NEG_POS * (misc[0, 1] / pos_cnt) + neg_sum / neg_cnt

    return jax.lax.cond(misc[0, 3] > 0.0, _atyp_path, lambda: scal[0, 0])


# speedup vs baseline: 858.3220x; 1.0292x over previous
import jax
import jax.numpy as jnp
from jax.experimental import pallas as pl
from jax.experimental.pallas import tpu as pltpu

B, C, H, W = 16, 4, 512, 512
N_HW = H * W
ROWS = 16
NCHUNK = H // ROWS
NEG_POS = 3
ONE_BITS = 0x3F800000


def _fold_lanes(x):
    return x[..., 0:128] + x[..., 128:256] + x[..., 256:384] + x[..., 384:512]


def _cumsum(x, axis):
    n = x.shape[axis]
    s = 1
    while s < n:
        pad = jnp.zeros_like(jax.lax.slice_in_dim(x, 0, s, axis=axis))
        shifted = jnp.concatenate(
            [pad, jax.lax.slice_in_dim(x, 0, n - s, axis=axis)], axis=axis)
        x = x + shifted
        s *= 2
    return x


def _main_body(yt_ref, yp_ref, yt_hbm, yp_hbm,
               scal_ref, pvec_ref, kf_ref, typb_ref, typm_ref, misc_ref,
               z_s, ps_s, acc_p, acc_all, rf_v, bidx_v, bidx_s,
               syt, syp, sem_b, sem_t, sem_p):
    j = pl.program_id(0)

    @pl.when(j == 0)
    def _():
        acc_p[...] = jnp.zeros_like(acc_p)
        acc_all[...] = jnp.zeros_like(acc_all)

    @pl.when(j < NCHUNK)
    def _():
        yt = yt_ref[...]
        yp = yp_ref[...]
        d = yt - yp
        d2 = d * d
        pos = yt >= 0.5
        posf = jnp.where(pos, 1.0, 0.0)
        zerof = jnp.where(d2 == 0.0, 1.0, posf)
        z_s[j] = jnp.sum(zerof, axis=(2, 3))
        ps_s[j] = jnp.sum(jnp.where(pos, d2, 0.0), axis=(2, 3))
        pf = _fold_lanes(posf)
        af = _fold_lanes(d2)
        acc_p[...] += pf[:, :, 0:ROWS // 2] + pf[:, :, ROWS // 2:ROWS]
        acc_all[...] += af[:, :, 0:ROWS // 2] + af[:, :, ROWS // 2:ROWS]

    @pl.when(j == NCHUNK - 1)
    def _():
        p = jnp.sum(acc_p[...], axis=(2, 3))
        alls = jnp.sum(acc_all[...], axis=(2, 3))
        zc = z_s[...]
        psc = ps_s[...]
        ps_tot = jnp.sum(psc, axis=0)
        z_tot = jnp.sum(zc, axis=0)
        k = jnp.minimum(3.0 * p, float(N_HW - 1))
        g = float(N_HW) - z_tot
        q = k - g
        typ = q >= 0.0
        nz_tot = alls - ps_tot
        cz = _cumsum(zc, axis=0)
        le = cz <= q[None]
        sum_full = jnp.sum(jnp.where(le, psc, 0.0), axis=0)
        bidx = jnp.sum(jnp.where(le, 1.0, 0.0), axis=0)
        cz_ex_b = jnp.max(jnp.where(le, cz, 0.0), axis=0)
        r = jnp.where(typ, q - cz_ex_b, 0.0)

        rf_v[...] = r
        kf_ref[...] = k
        typb_ref[...] = jnp.where(typ, 0.5 * (nz_tot + sum_full), 0.0)
        typm_ref[...] = jnp.where(typ, 1.0, 0.0)

        row = jax.lax.broadcasted_iota(jnp.int32, (B, C), 0)
        col = jax.lax.broadcasted_iota(jnp.int32, (B, C), 1)
        misc = jnp.where(jnp.logical_and(row == 0, col == 0), jnp.sum(p), 0.0)
        misc = jnp.where(jnp.logical_and(row == 0, col == 1),
                         0.5 * jnp.sum(ps_tot), misc)
        misc = jnp.where(jnp.logical_and(row == 0, col == 2),
                         jnp.sum(k), misc)
        misc = jnp.where(jnp.logical_and(row == 0, col == 3),
                         jnp.sum(jnp.where(typ, 0.0, 1.0)), misc)
        misc_ref[...] = misc

        bidx_v[...] = bidx.astype(jnp.int32)
        cp = pltpu.make_async_copy(bidx_v, bidx_s, sem_b)
        cp.start()
        cp.wait()
        for m in range(B * C):
            b = m // C
            c = m % C
            off = bidx_s[b, c] * ROWS
            pltpu.make_async_copy(
                yt_hbm.at[b, c, pl.ds(off, ROWS), :], syt.at[b, c],
                sem_t.at[b, c]).start()
            pltpu.make_async_copy(
                yp_hbm.at[b, c, pl.ds(off, ROWS), :], syp.at[b, c],
                sem_p.at[b, c]).start()

    @pl.when(j == NCHUNK)
    def _():
        for m in range(B * C):
            b = m // C
            c = m % C
            off = bidx_s[b, c] * ROWS
            pltpu.make_async_copy(
                yt_hbm.at[b, c, pl.ds(off, ROWS), :], syt.at[b, c],
                sem_t.at[b, c]).wait()
            pltpu.make_async_copy(
                yp_hbm.at[b, c, pl.ds(off, ROWS), :], syp.at[b, c],
                sem_p.at[b, c]).wait()

        yt = syt[...]
        yp = syp[...]
        d = yt - yp
        d2 = d * d
        pos = yt >= 0.5
        zero = jnp.logical_or(pos, d2 == 0.0)
        zf = jnp.where(zero, 1.0, 0.0)
        cs = _cumsum(zf, axis=3)
        rowtot = cs[:, :, :, W - 1:W]
        rowcum = _cumsum(rowtot, axis=2) - rowtot
        crank = cs + rowcum
        rv = rf_v[...][:, :, None, None]
        sel = jnp.logical_and(zero, crank <= rv)
        pvec = 0.5 * jnp.sum(jnp.where(sel, d2, 0.0), axis=(2, 3))
        pvec_ref[...] = pvec

        misc = misc_ref[...]
        neg_sum = jnp.sum(typb_ref[...] + pvec)
        pos_cnt = jnp.maximum(misc[0, 0], 1.0)
        neg_cnt = jnp.maximum(misc[0, 2], 1.0)
        out = NEG_POS * (misc[0, 1] / pos_cnt) + neg_sum / neg_cnt
        scal_ref[...] = jnp.full((1, 1), 1.0) * out


def _count_body(t_ref, yt_ref, yp_ref, cnt_ref):
    j = pl.program_id(0)

    @pl.when(j == 0)
    def _():
        cnt_ref[...] = jnp.zeros_like(cnt_ref)

    yt = yt_ref[...]
    yp = yp_ref[...]
    pos = yt >= 0.5
    loss = jnp.where(pos, 0.0, jnp.abs(yt - yp))
    t = t_ref[...][:, :, None, None]
    cnt_ref[...] += jnp.sum((loss > t).astype(jnp.float32), axis=(2, 3))


def _gt_body(t_ref, yt_ref, yp_ref, cnt_ref, sum_ref):
    j = pl.program_id(0)

    @pl.when(j == 0)
    def _():
        cnt_ref[...] = jnp.zeros_like(cnt_ref)
        sum_ref[...] = jnp.zeros_like(sum_ref)

    yt = yt_ref[...]
    yp = yp_ref[...]
    d = yt - yp
    pos = yt >= 0.5
    loss = jnp.where(pos, 0.0, jnp.abs(d))
    t = t_ref[...][:, :, None, None]
    gt = loss > t
    cnt_ref[...] += jnp.sum(gt.astype(jnp.float32), axis=(2, 3))
    sum_ref[...] += jnp.sum(jnp.where(gt, 0.5 * d * d, 0.0), axis=(2, 3))


_STATS_BLOCK = pl.BlockSpec(
    (B, C, ROWS, W), lambda j: (0, 0, jnp.minimum(j, NCHUNK - 1), 0))
_BLOCK4D = pl.BlockSpec((B, C, ROWS, W), lambda j: (0, 0, j, 0))
_BC_IN = pl.BlockSpec((B, C), lambda j: (0, 0))
_BC_OUT = pl.BlockSpec((B, C), lambda j: (0, 0))


def _count_gt(yt, yp, t):
    return pl.pallas_call(
        _count_body,
        grid=(NCHUNK,),
        in_specs=[_BC_IN, _BLOCK4D, _BLOCK4D],
        out_specs=_BC_OUT,
        out_shape=jax.ShapeDtypeStruct((B, C), jnp.float32),
    )(t, yt, yp)


def _atypical_sums(yt, yp, kf):
    k = kf.astype(jnp.int32)

    def body(_, carry):
        lo, hi = carry
        mid = jnp.where(lo < hi, (lo + hi) // 2, lo)
        t = jax.lax.bitcast_convert_type(mid, jnp.float32)
        cnt = _count_gt(yt, yp, t)
        less = cnt < kf
        lo2 = jnp.where(jnp.logical_and(lo < hi, jnp.logical_not(less)),
                        mid + 1, lo)
        hi2 = jnp.where(jnp.logical_and(lo < hi, less), mid, hi)
        return lo2, hi2

    lo = jnp.zeros_like(k)
    hi = jnp.full_like(k, ONE_BITS)
    lo, _ = jax.lax.fori_loop(0, 31, body, (lo, hi))
    t = jax.lax.bitcast_convert_type(lo, jnp.float32)
    cnt, gsum = pl.pallas_call(
        _gt_body,
        grid=(NCHUNK,),
        in_specs=[_BC_IN, _BLOCK4D, _BLOCK4D],
        out_specs=[_BC_OUT, _BC_OUT],
        out_shape=[jax.ShapeDtypeStruct((B, C), jnp.float32),
                   jax.ShapeDtypeStruct((B, C), jnp.float32)],
    )(t, yt, yp)
    sl1_t = jnp.where(t < 1.0, 0.5 * t * t, t - 0.5)
    return gsum + (kf - cnt) * sl1_t


def kernel(y_true, y_pred):
    yt = y_true
    yp = y_pred

    scal, pvec, kf, typb, typm, misc = pl.pallas_call(
        _main_body,
        grid=(NCHUNK + 1,),
        in_specs=[_STATS_BLOCK, _STATS_BLOCK,
                  pl.BlockSpec(memory_space=pl.ANY),
                  pl.BlockSpec(memory_space=pl.ANY)],
        out_specs=[pl.BlockSpec((1, 1), lambda j: (0, 0))] + [_BC_OUT] * 5,
        out_shape=[jax.ShapeDtypeStruct((1, 1), jnp.float32)] +
                  [jax.ShapeDtypeStruct((B, C), jnp.float32)] * 5,
        scratch_shapes=[
            pltpu.VMEM((NCHUNK, B, C), jnp.float32),
            pltpu.VMEM((NCHUNK, B, C), jnp.float32),
            pltpu.VMEM((B, C, ROWS // 2, 128), jnp.float32),
            pltpu.VMEM((B, C, ROWS // 2, 128), jnp.float32),
            pltpu.VMEM((B, C), jnp.float32),
            pltpu.VMEM((B, C), jnp.int32),
            pltpu.SMEM((B, C), jnp.int32),
            pltpu.VMEM((B, C, ROWS, W), jnp.float32),
            pltpu.VMEM((B, C, ROWS, W), jnp.float32),
            pltpu.SemaphoreType.DMA,
            pltpu.SemaphoreType.DMA((B, C)),
            pltpu.SemaphoreType.DMA((B, C)),
        ],
    )(yt, yp, yt, yp)

    def _atyp_path():
        atyp = _atypical_sums(yt, yp, kf)
        neg_sum = jnp.sum(jnp.where(typm > 0.0, typb + pvec, atyp))
        pos_cnt = jnp.maximum(misc[0, 0], 1.0)
        neg_cnt = jnp.maximum(misc[0, 2], 1.0)
        return NEG_POS * (misc[0, 1] / pos_cnt) + neg_sum / neg_cnt

    return jax.lax.cond(misc[0, 3] > 0.0, _atyp_path, lambda: scal[0, 0])
